# baseline (speedup 1.0000x reference)
import functools

import jax
import jax.numpy as jnp
from jax import lax
from jax.experimental import pallas as pl
from jax.experimental.pallas import tpu as pltpu
from jax.experimental.pallas import tpu_sc as plsc

NC = 2
NS = 16
L = 16


def _make_sc_kernel(M, D, B):
    total = M * D
    U = B * D
    NCHUNK = 4
    assert total % NCHUNK == 0
    CW = total // NCHUNK
    NPASS = NCHUNK // NC
    PW = CW // NS
    assert CW % NS == 0 and PW % 8 == 0
    UPT = U // NS
    BLK = 2048
    assert UPT % BLK == 0
    NBLK = UPT // BLK
    NVEC = BLK // L
    SW = 20000
    assert PW % SW == 0 and SW % 8 == 0
    NSTAGE = PW // SW

    mesh = plsc.VectorSubcoreMesh(core_axis_name="c", subcore_axis_name="s")

    @functools.partial(
        pl.kernel,
        mesh=mesh,
        out_type=jax.ShapeDtypeStruct((total,), jnp.float32),
        scratch_types=[
            pltpu.VMEM_SHARED((CW + 16,), jnp.float32),
            pltpu.VMEM((BLK,), jnp.int32),
            pltpu.VMEM((BLK,), jnp.float32),
            pltpu.VMEM((BLK,), jnp.int32),
            pltpu.VMEM((SW,), jnp.float32),
            pltpu.SemaphoreType.DMA,
        ],
    )
    def scatter_add_kernel(x_hbm, idx_hbm, src_hbm, out_hbm,
                           accum, idx_raw, src_buf, idx_scat, stage, sem):
        c = lax.axis_index("c")
        s = lax.axis_index("s")
        iota = lax.iota(jnp.int32, L)

        for p in range(NPASS):
            base = (p * NC + c) * CW

            def init_body(t, _):
                pltpu.sync_copy(x_hbm.at[pl.ds(base + s * PW + t * SW, SW)],
                                stage)
                pltpu.sync_copy(stage, accum.at[pl.ds(s * PW + t * SW, SW)])
                return 0

            lax.fori_loop(0, NSTAGE, init_body, 0)
            plsc.subcore_barrier()

            def block_body(b, _):
                off = s * UPT + b * BLK
                pltpu.sync_copy(idx_hbm.at[pl.ds(off, BLK)], idx_raw)
                pltpu.sync_copy(src_hbm.at[pl.ds(off, BLK)], src_buf)

                def vec_body(i, _):
                    v = idx_raw[pl.ds(i * L, L)]
                    col = iota + (i & 3) * L
                    rel = v * D + col - base
                    ok = (rel >= 0) & (rel < CW)
                    idx_scat[pl.ds(i * L, L)] = jnp.where(ok, rel, CW)
                    return 0

                lax.fori_loop(0, NVEC, vec_body, 0)
                pltpu.async_copy(src_buf, accum.at[idx_scat], sem,
                                 add=True).wait()
                return 0

            lax.fori_loop(0, NBLK, block_body, 0)
            plsc.subcore_barrier()

            def wb_body(t, _):
                pltpu.sync_copy(accum.at[pl.ds(s * PW + t * SW, SW)], stage)
                pltpu.sync_copy(stage,
                                out_hbm.at[pl.ds(base + s * PW + t * SW, SW)])
                return 0

            lax.fori_loop(0, NSTAGE, wb_body, 0)
            plsc.subcore_barrier()

    return scatter_add_kernel


def kernel(x, dim, index, src, out):
    M, D = x.shape
    B = src.shape[0]
    del out
    rows = index + jnp.asarray(dim, dtype=index.dtype)
    sc = _make_sc_kernel(M, D, B)
    res = sc(x.reshape(-1), rows.reshape(-1), src.reshape(-1))
    return res.reshape(M, D)


# speedup vs baseline: 11.2962x; 11.2962x over previous
import functools

import jax
import jax.numpy as jnp
from jax import lax
from jax.experimental import pallas as pl
from jax.experimental.pallas import tpu as pltpu
from jax.experimental.pallas import tpu_sc as plsc

NC = 2
NS = 16
L = 16


def _make_sc_kernel(M, D, B):
    total = M * D
    NCHUNK = 4
    assert D == NCHUNK * NS
    CW = NS * M
    NPASS = NCHUNK // NC
    PW = CW // NS
    assert PW % 8 == 0
    BLK = 2048
    assert B % BLK == 0
    NBLK = B // BLK
    NVEC = BLK // L
    SW = 10000
    assert PW % SW == 0 and SW % 8 == 0
    NSTAGE = PW // SW

    mesh = plsc.VectorSubcoreMesh(core_axis_name="c", subcore_axis_name="s")

    @functools.partial(
        pl.kernel,
        mesh=mesh,
        out_type=jax.ShapeDtypeStruct((total,), jnp.float32),
        scratch_types=[
            pltpu.VMEM_SHARED((CW + 16,), jnp.float32),
            pltpu.VMEM((BLK,), jnp.int32),
            pltpu.VMEM((BLK,), jnp.int32),
            pltpu.VMEM((BLK,), jnp.float32),
            pltpu.VMEM((BLK,), jnp.float32),
            pltpu.VMEM((BLK,), jnp.int32),
            pltpu.VMEM((BLK,), jnp.int32),
            pltpu.VMEM((SW,), jnp.float32),
            pltpu.SemaphoreType.DMA,
        ],
    )
    def scatter_add_kernel(xt_hbm, idxt_hbm, srct_hbm, outt_hbm,
                           accum, idx_raw0, idx_raw1, src_buf0, src_buf1,
                           idx_scat0, idx_scat1, stage, sem):
        idx_raw = (idx_raw0, idx_raw1)
        src_buf = (src_buf0, src_buf1)
        idx_scat = (idx_scat0, idx_scat1)
        c = lax.axis_index("c")
        s = lax.axis_index("s")

        for p in range(NPASS):
            k = p * NC + c
            base = k * CW
            colbase = (k * NS + s) * B

            def init_body(t, _):
                pltpu.sync_copy(xt_hbm.at[pl.ds(base + s * PW + t * SW, SW)],
                                stage)
                pltpu.sync_copy(stage, accum.at[pl.ds(s * PW + t * SW, SW)])
                return 0

            lax.fori_loop(0, NSTAGE, init_body, 0)
            plsc.subcore_barrier()

            for b in range(NBLK):
                d = b % 2
                pltpu.sync_copy(idxt_hbm.at[pl.ds(colbase + b * BLK, BLK)],
                                idx_raw[d])
                pltpu.sync_copy(srct_hbm.at[pl.ds(colbase + b * BLK, BLK)],
                                src_buf[d])

                def vec_body(i, _, d=d):
                    v = idx_raw[d][pl.ds(i * L, L)]
                    idx_scat[d][pl.ds(i * L, L)] = v + s * M
                    return 0

                lax.fori_loop(0, NVEC, vec_body, 0)
                if b >= 1:
                    pltpu.make_async_copy(src_buf[1 - d],
                                          accum.at[idx_scat[1 - d]],
                                          sem).wait()
                pltpu.async_copy(src_buf[d], accum.at[idx_scat[d]],
                                 sem, add=True)
            pltpu.make_async_copy(src_buf[(NBLK - 1) % 2],
                                  accum.at[idx_scat[(NBLK - 1) % 2]],
                                  sem).wait()
            plsc.subcore_barrier()

            def wb_body(t, _):
                pltpu.sync_copy(accum.at[pl.ds(s * PW + t * SW, SW)], stage)
                pltpu.sync_copy(stage,
                                outt_hbm.at[pl.ds(base + s * PW + t * SW, SW)])
                return 0

            lax.fori_loop(0, NSTAGE, wb_body, 0)
            plsc.subcore_barrier()

    return scatter_add_kernel


def kernel(x, dim, index, src, out):
    M, D = x.shape
    B = src.shape[0]
    del out
    rows = index + jnp.asarray(dim, dtype=index.dtype)
    sc = _make_sc_kernel(M, D, B)
    outt = sc(x.T.reshape(-1), rows.T.reshape(-1), src.T.reshape(-1))
    return outt.reshape(D, M).T
